# Initial kernel scaffold; baseline (speedup 1.0000x reference)
import functools

import jax
import jax.numpy as jnp
from jax import lax
from jax.experimental import pallas as pl
from jax.experimental.pallas import tpu as pltpu
from jax.experimental.pallas import tpu_sc as plsc

NC = 2
NS = 16
NW = NC * NS
C = 80
D = 128
ROWBLK = 1024


def _sc_mesh():
    return plsc.VectorSubcoreMesh(core_axis_name="c", subcore_axis_name="s")


def _make_deg_kernel(npad, chunks):
    n_per_tile = npad // NS

    @functools.partial(
        pl.kernel,
        out_type=jax.ShapeDtypeStruct((NC * npad,), jnp.float32),
        mesh=_sc_mesh(),
        scratch_types=[
            pltpu.VMEM((chunks, C), jnp.int32),
            pltpu.VMEM((chunks, C), jnp.float32),
            pltpu.VMEM((n_per_tile,), jnp.float32),
            pltpu.VMEM_SHARED((npad,), jnp.float32),
        ],
    )
    def deg_kernel(dst_hbm, w_hbm, out_hbm, dst_v, w_v, zbuf, acc):
        c = lax.axis_index("c")
        s = lax.axis_index("s")
        b = c * NS + s

        pltpu.sync_copy(dst_hbm.at[b], dst_v)
        pltpu.sync_copy(w_hbm.at[b], w_v)

        def zstore(i, _):
            zbuf[pl.ds(i * 16, 16)] = jnp.zeros((16,), jnp.float32)
            return 0
        lax.fori_loop(0, n_per_tile // 16, zstore, 0)
        pltpu.sync_copy(zbuf, acc.at[pl.ds(s * n_per_tile, n_per_tile)])
        plsc.subcore_barrier()

        def chunk(g, _):
            pltpu.sync_copy(w_v.at[g], acc.at[dst_v.at[g]], add=True)
            return 0
        lax.fori_loop(0, chunks, chunk, 0)
        plsc.subcore_barrier()

        pltpu.sync_copy(acc.at[pl.ds(s * n_per_tile, n_per_tile)],
                        out_hbm.at[pl.ds(c * npad + s * n_per_tile, n_per_tile)])

    return deg_kernel


def _make_agg_kernel(npad, chunks):
    n_per_tile = npad // NS
    zrows = 64

    @functools.partial(
        pl.kernel,
        out_type=jax.ShapeDtypeStruct((NC * npad, D), jnp.float32),
        mesh=_sc_mesh(),
        scratch_types=[
            pltpu.VMEM((chunks, C), jnp.int32),
            pltpu.VMEM((chunks, C), jnp.int32),
            pltpu.VMEM((chunks, C), jnp.float32),
            pltpu.VMEM((npad,), jnp.float32),
            pltpu.VMEM((C,), jnp.float32),
            pltpu.VMEM((C, D), jnp.float32),
            pltpu.VMEM((zrows, D), jnp.float32),
            pltpu.VMEM_SHARED((npad, D), jnp.float32),
            pltpu.SemaphoreType.DMA,
        ],
    )
    def agg_kernel(y_hbm, src_hbm, dst_hbm, w_hbm, dinv_hbm, out_hbm,
                   src_v, dst_v, w_v, dinv_v, norm_v, rows_v, zbuf, acc, sem):
        c = lax.axis_index("c")
        s = lax.axis_index("s")
        b = c * NS + s

        pltpu.sync_copy(src_hbm.at[b], src_v)
        pltpu.sync_copy(dst_hbm.at[b], dst_v)
        pltpu.sync_copy(w_hbm.at[b], w_v)
        pltpu.sync_copy(dinv_hbm, dinv_v)

        def zstore(i, _):
            for j in range(D // 16):
                zbuf[i, pl.ds(j * 16, 16)] = jnp.zeros((16,), jnp.float32)
            return 0
        lax.fori_loop(0, zrows, zstore, 0)

        def zcopy(k, _):
            pltpu.sync_copy(
                zbuf, acc.at[pl.ds(s * n_per_tile + k * zrows, zrows)])
            return 0
        lax.fori_loop(0, n_per_tile // zrows, zcopy, 0)
        plsc.subcore_barrier()

        def chunk(g, _):
            pltpu.async_copy(y_hbm.at[src_v.at[g]], rows_v, sem).wait()
            for k in range(C // 16):
                s16 = src_v[g, pl.ds(k * 16, 16)]
                d16 = dst_v[g, pl.ds(k * 16, 16)]
                w16 = w_v[g, pl.ds(k * 16, 16)]
                n16 = (plsc.load_gather(dinv_v, [s16]) * w16 *
                       plsc.load_gather(dinv_v, [d16]))
                norm_v[pl.ds(k * 16, 16)] = n16

            def edge(e, _):
                nv = norm_v[e]
                for j in range(D // 16):
                    rows_v[e, pl.ds(j * 16, 16)] = (
                        rows_v[e, pl.ds(j * 16, 16)] * nv)
                return 0
            lax.fori_loop(0, C, edge, 0)

            pltpu.sync_copy(rows_v, acc.at[dst_v.at[g]], add=True)
            return 0
        lax.fori_loop(0, chunks, chunk, 0)
        plsc.subcore_barrier()

        pltpu.sync_copy(acc.at[pl.ds(s * n_per_tile, n_per_tile)],
                        out_hbm.at[pl.ds(c * npad + s * n_per_tile, n_per_tile)])

    return agg_kernel


def _make_prep1(npad):
    grid = npad // ROWBLK

    def body(degp_ref, x_ref, w1_ref, dinv_ref, y1_ref):
        deg = degp_ref[0] + degp_ref[1]
        dinv_ref[...] = jnp.where(deg > 0, lax.rsqrt(deg), 0.0)
        y1_ref[...] = jnp.dot(x_ref[...], w1_ref[...],
                              preferred_element_type=jnp.float32)

    return pl.pallas_call(
        body,
        grid=(grid,),
        in_specs=[
            pl.BlockSpec((NC, ROWBLK // D, D), lambda i: (0, i, 0)),
            pl.BlockSpec((ROWBLK, D), lambda i: (i, 0)),
            pl.BlockSpec((D, D), lambda i: (0, 0)),
        ],
        out_specs=[
            pl.BlockSpec((ROWBLK // D, D), lambda i: (i, 0)),
            pl.BlockSpec((ROWBLK, D), lambda i: (i, 0)),
        ],
        out_shape=[
            jax.ShapeDtypeStruct((npad // D, D), jnp.float32),
            jax.ShapeDtypeStruct((npad, D), jnp.float32),
        ],
    )


def _make_layer2(npad):
    grid = npad // ROWBLK

    def body(aggp_ref, b1_ref, w2_ref, y2_ref):
        h = jax.nn.relu(aggp_ref[0] + aggp_ref[1] + b1_ref[...])
        y2_ref[...] = jnp.dot(h, w2_ref[...],
                              preferred_element_type=jnp.float32)

    return pl.pallas_call(
        body,
        grid=(grid,),
        in_specs=[
            pl.BlockSpec((NC, ROWBLK, D), lambda i: (0, i, 0)),
            pl.BlockSpec((1, D), lambda i: (0, 0)),
            pl.BlockSpec((D, D), lambda i: (0, 0)),
        ],
        out_specs=pl.BlockSpec((ROWBLK, D), lambda i: (i, 0)),
        out_shape=jax.ShapeDtypeStruct((npad, D), jnp.float32),
    )


def _make_head(npad):
    grid = npad // ROWBLK

    def body(aggp_ref, b2_ref, wp1_ref, bp1_ref, wp2_ref, bp2_ref,
             emb_ref, z_ref):
        emb = aggp_ref[0] + aggp_ref[1] + b2_ref[...]
        emb_ref[...] = emb
        t = jax.nn.relu(jnp.dot(emb, wp1_ref[...],
                                preferred_element_type=jnp.float32)
                        + bp1_ref[...])
        z_ref[...] = jnp.dot(t, wp2_ref[...],
                             preferred_element_type=jnp.float32) + bp2_ref[...]

    return pl.pallas_call(
        body,
        grid=(grid,),
        in_specs=[
            pl.BlockSpec((NC, ROWBLK, D), lambda i: (0, i, 0)),
            pl.BlockSpec((1, D), lambda i: (0, 0)),
            pl.BlockSpec((D, D), lambda i: (0, 0)),
            pl.BlockSpec((1, D), lambda i: (0, 0)),
            pl.BlockSpec((D, D), lambda i: (0, 0)),
            pl.BlockSpec((1, D), lambda i: (0, 0)),
        ],
        out_specs=[
            pl.BlockSpec((ROWBLK, D), lambda i: (i, 0)),
            pl.BlockSpec((ROWBLK, D), lambda i: (i, 0)),
        ],
        out_shape=[
            jax.ShapeDtypeStruct((npad, D), jnp.float32),
            jax.ShapeDtypeStruct((npad, D), jnp.float32),
        ],
    )


def kernel(x, edge_index, edge_weight, W1, b1, W2, b2, Wp1, bp1, Wp2, bp2):
    n = x.shape[0]
    e = edge_index.shape[1]

    npad = -(-n // ROWBLK) * ROWBLK
    e2 = e + n
    chunks = -(-e2 // (NW * C))
    e2p = chunks * NW * C

    loop = jnp.arange(n, dtype=jnp.int32)
    src2 = jnp.concatenate(
        [edge_index[0], loop,
         jnp.zeros((e2p - e2,), jnp.int32)]).reshape(NW, chunks, C)
    dst2 = jnp.concatenate(
        [edge_index[1], loop,
         jnp.zeros((e2p - e2,), jnp.int32)]).reshape(NW, chunks, C)
    w2 = jnp.concatenate(
        [edge_weight, jnp.ones((n,), jnp.float32),
         jnp.zeros((e2p - e2,), jnp.float32)]).reshape(NW, chunks, C)

    xp = jnp.pad(x, ((0, npad - n), (0, 0)))
    b1r = b1.reshape(1, D)
    b2r = b2.reshape(1, D)
    bp1r = bp1.reshape(1, D)
    bp2r = bp2.reshape(1, D)

    deg_p = _make_deg_kernel(npad, chunks)(dst2, w2).reshape(NC, npad // D, D)
    dinv2d, y1 = _make_prep1(npad)(deg_p, xp, W1)
    dinv = dinv2d.reshape(npad)

    agg = _make_agg_kernel(npad, chunks)
    agg1 = agg(y1, src2, dst2, w2, dinv).reshape(NC, npad, D)
    y2 = _make_layer2(npad)(agg1, b1r, W2)
    agg2 = agg(y2, src2, dst2, w2, dinv).reshape(NC, npad, D)
    emb, z = _make_head(npad)(agg2, b2r, Wp1, bp1r, Wp2, bp2r)

    return (emb[:n], z[:n])


# speedup vs baseline: 7.5401x; 7.5401x over previous
import functools

import jax
import jax.numpy as jnp
from jax import lax
from jax.experimental import pallas as pl
from jax.experimental.pallas import tpu as pltpu
from jax.experimental.pallas import tpu_sc as plsc

NC = 2
NS = 16
NW = NC * NS
C = 80
D = 128
ROWBLK = 1024


def _sc_mesh():
    return plsc.VectorSubcoreMesh(core_axis_name="c", subcore_axis_name="s")


def _make_deg_kernel(npad, chunks):
    n_per_tile = npad // NS

    @functools.partial(
        pl.kernel,
        out_type=jax.ShapeDtypeStruct((NC * npad,), jnp.float32),
        mesh=_sc_mesh(),
        scratch_types=[
            pltpu.VMEM((chunks, C), jnp.int32),
            pltpu.VMEM((chunks, C), jnp.float32),
            pltpu.VMEM((n_per_tile,), jnp.float32),
            pltpu.VMEM_SHARED((npad,), jnp.float32),
        ],
        compiler_params=pltpu.CompilerParams(
            needs_layout_passes=False, use_tc_tiling_on_sc=False),
    )
    def deg_kernel(dst_hbm, w_hbm, out_hbm, dst_v, w_v, zbuf, acc):
        c = lax.axis_index("c")
        s = lax.axis_index("s")
        b = c * NS + s

        pltpu.sync_copy(dst_hbm.at[b], dst_v)
        pltpu.sync_copy(w_hbm.at[b], w_v)

        def zstore(i, _):
            zbuf[pl.ds(i * 16, 16)] = jnp.zeros((16,), jnp.float32)
            return 0
        lax.fori_loop(0, n_per_tile // 16, zstore, 0)
        pltpu.sync_copy(zbuf, acc.at[pl.ds(s * n_per_tile, n_per_tile)])
        plsc.subcore_barrier()

        def chunk(g, _):
            pltpu.sync_copy(w_v.at[g], acc.at[dst_v.at[g]], add=True)
            return 0
        lax.fori_loop(0, chunks, chunk, 0)
        plsc.subcore_barrier()

        pltpu.sync_copy(acc.at[pl.ds(s * n_per_tile, n_per_tile)],
                        out_hbm.at[pl.ds(c * npad + s * n_per_tile, n_per_tile)])

    return deg_kernel


def _make_agg_kernel(npad, nacc, chunks):
    n_per_tile = nacc // NS
    dq = D // 4
    zrows = n_per_tile // 5

    @functools.partial(
        pl.kernel,
        out_type=jax.ShapeDtypeStruct((4 * nacc, dq), jnp.float32),
        mesh=_sc_mesh(),
        scratch_types=[
            pltpu.VMEM((chunks, C), jnp.int32),
            pltpu.VMEM((chunks, C), jnp.int32),
            pltpu.VMEM((chunks, C), jnp.float32),
            pltpu.VMEM((chunks, C), jnp.int32),
            pltpu.VMEM((npad,), jnp.float32),
            pltpu.VMEM((C, dq), jnp.float32),
            pltpu.VMEM((zrows, dq), jnp.float32),
            pltpu.VMEM_SHARED((nacc, dq), jnp.float32),
            pltpu.SemaphoreType.DMA,
        ],
        compiler_params=pltpu.CompilerParams(
            needs_layout_passes=False, use_tc_tiling_on_sc=False),
    )
    def agg_kernel(yq_hbm, src_hbm, dst_hbm, w_hbm, dinv_hbm, out_hbm,
                   src_v, dst_v, w_v, srcq_v, dinv_v, rows_v, zbuf, acc, sem):
        c = lax.axis_index("c")
        s = lax.axis_index("s")

        pltpu.sync_copy(src_hbm.at[s], src_v)
        pltpu.sync_copy(dst_hbm.at[s], dst_v)
        pltpu.sync_copy(w_hbm.at[s], w_v)
        pltpu.sync_copy(dinv_hbm, dinv_v)

        def zstore(i, _):
            for j in range(dq // 16):
                zbuf[i, pl.ds(j * 16, 16)] = jnp.zeros((16,), jnp.float32)
            return 0
        lax.fori_loop(0, zrows, zstore, 0)

        for p in range(2):
            qoff = (2 * p + c) * npad

            def shift(g, _):
                for k in range(C // 16):
                    srcq_v[g, pl.ds(k * 16, 16)] = (
                        src_v[g, pl.ds(k * 16, 16)] + qoff)
                return 0
            lax.fori_loop(0, chunks, shift, 0)

            def zcopy(k, _):
                pltpu.sync_copy(
                    zbuf, acc.at[pl.ds(s * n_per_tile + k * zrows, zrows)])
                return 0
            lax.fori_loop(0, 5, zcopy, 0)
            plsc.subcore_barrier()

            def chunk(g, _):
                pltpu.async_copy(yq_hbm.at[srcq_v.at[g]], rows_v, sem).wait()

                def group(k, _):
                    s16 = src_v[g, pl.ds(k * 16, 16)]
                    d16 = dst_v[g, pl.ds(k * 16, 16)]
                    w16 = w_v[g, pl.ds(k * 16, 16)]
                    n16 = (plsc.load_gather(dinv_v, [s16]) * w16 *
                           plsc.load_gather(dinv_v, [d16]))
                    for i in range(16):
                        nv = n16[i]
                        row = k * 16 + i
                        for j in range(dq // 16):
                            rows_v[row, pl.ds(j * 16, 16)] = (
                                rows_v[row, pl.ds(j * 16, 16)] * nv)
                    return 0
                lax.fori_loop(0, C // 16, group, 0)

                pltpu.sync_copy(rows_v, acc.at[dst_v.at[g]], add=True)
                return 0
            lax.fori_loop(0, chunks, chunk, 0)
            plsc.subcore_barrier()

            pltpu.sync_copy(
                acc.at[pl.ds(s * n_per_tile, n_per_tile)],
                out_hbm.at[pl.ds((2 * p + c) * nacc + s * n_per_tile,
                                 n_per_tile)])
            plsc.subcore_barrier()

    return agg_kernel


def _make_prep1(npad):
    grid = npad // ROWBLK

    def body(degp_ref, x_ref, w1_ref, dinv_ref, y1_ref):
        deg = degp_ref[0] + degp_ref[1]
        dinv_ref[...] = jnp.where(deg > 0, lax.rsqrt(deg), 0.0)
        y1_ref[...] = jnp.dot(x_ref[...], w1_ref[...],
                              preferred_element_type=jnp.float32)

    return pl.pallas_call(
        body,
        grid=(grid,),
        in_specs=[
            pl.BlockSpec((NC, ROWBLK // D, D), lambda i: (0, i, 0)),
            pl.BlockSpec((ROWBLK, D), lambda i: (i, 0)),
            pl.BlockSpec((D, D), lambda i: (0, 0)),
        ],
        out_specs=[
            pl.BlockSpec((ROWBLK // D, D), lambda i: (i, 0)),
            pl.BlockSpec((ROWBLK, D), lambda i: (i, 0)),
        ],
        out_shape=[
            jax.ShapeDtypeStruct((npad // D, D), jnp.float32),
            jax.ShapeDtypeStruct((npad, D), jnp.float32),
        ],
    )


def _make_layer2(npad):
    grid = npad // ROWBLK

    def body(agg_ref, b1_ref, w2_ref, y2_ref):
        h = jax.nn.relu(agg_ref[...] + b1_ref[...])
        y2_ref[...] = jnp.dot(h, w2_ref[...],
                              preferred_element_type=jnp.float32)

    return pl.pallas_call(
        body,
        grid=(grid,),
        in_specs=[
            pl.BlockSpec((ROWBLK, D), lambda i: (i, 0)),
            pl.BlockSpec((1, D), lambda i: (0, 0)),
            pl.BlockSpec((D, D), lambda i: (0, 0)),
        ],
        out_specs=pl.BlockSpec((ROWBLK, D), lambda i: (i, 0)),
        out_shape=jax.ShapeDtypeStruct((npad, D), jnp.float32),
    )


def _make_head(npad):
    grid = npad // ROWBLK

    def body(agg_ref, b2_ref, wp1_ref, bp1_ref, wp2_ref, bp2_ref,
             emb_ref, z_ref):
        emb = agg_ref[...] + b2_ref[...]
        emb_ref[...] = emb
        t = jax.nn.relu(jnp.dot(emb, wp1_ref[...],
                                preferred_element_type=jnp.float32)
                        + bp1_ref[...])
        z_ref[...] = jnp.dot(t, wp2_ref[...],
                             preferred_element_type=jnp.float32) + bp2_ref[...]

    return pl.pallas_call(
        body,
        grid=(grid,),
        in_specs=[
            pl.BlockSpec((ROWBLK, D), lambda i: (i, 0)),
            pl.BlockSpec((1, D), lambda i: (0, 0)),
            pl.BlockSpec((D, D), lambda i: (0, 0)),
            pl.BlockSpec((1, D), lambda i: (0, 0)),
            pl.BlockSpec((D, D), lambda i: (0, 0)),
            pl.BlockSpec((1, D), lambda i: (0, 0)),
        ],
        out_specs=[
            pl.BlockSpec((ROWBLK, D), lambda i: (i, 0)),
            pl.BlockSpec((ROWBLK, D), lambda i: (i, 0)),
        ],
        out_shape=[
            jax.ShapeDtypeStruct((npad, D), jnp.float32),
            jax.ShapeDtypeStruct((npad, D), jnp.float32),
        ],
    )


def kernel(x, edge_index, edge_weight, W1, b1, W2, b2, Wp1, bp1, Wp2, bp2):
    n = x.shape[0]
    e = edge_index.shape[1]

    npad = -(-n // ROWBLK) * ROWBLK
    e2 = e + n
    chunks = -(-e2 // (NW * C))
    chunks2 = -(-e2 // (NS * C))
    e2p = chunks2 * NS * C
    assert chunks * NW * C == e2p

    loop = jnp.arange(n, dtype=jnp.int32)
    pad_i = jnp.zeros((e2p - e2,), jnp.int32)
    pad_f = jnp.zeros((e2p - e2,), jnp.float32)
    src_flat = jnp.concatenate([edge_index[0], loop, pad_i])
    dst_flat = jnp.concatenate([edge_index[1], loop, pad_i])
    w_flat = jnp.concatenate([edge_weight, jnp.ones((n,), jnp.float32), pad_f])

    dst2 = dst_flat.reshape(NW, chunks, C)
    w2 = w_flat.reshape(NW, chunks, C)
    srcA = src_flat.reshape(NS, chunks2, C)
    dstA = dst_flat.reshape(NS, chunks2, C)
    wA = w_flat.reshape(NS, chunks2, C)

    xp = jnp.pad(x, ((0, npad - n), (0, 0)))
    b1r = b1.reshape(1, D)
    b2r = b2.reshape(1, D)
    bp1r = bp1.reshape(1, D)
    bp2r = bp2.reshape(1, D)

    deg_p = _make_deg_kernel(npad, chunks)(dst2, w2).reshape(NC, npad // D, D)
    dinv2d, y1 = _make_prep1(npad)(deg_p, xp, W1)
    dinv = dinv2d.reshape(npad)

    dq = D // 4
    agg = _make_agg_kernel(npad, n, chunks2)

    def run_agg(y):
        yq = jnp.concatenate([y[:, i * dq:(i + 1) * dq] for i in range(4)],
                             axis=0)
        out = agg(yq, srcA, dstA, wA, dinv).reshape(4, n, dq)
        full = jnp.concatenate([out[0], out[1], out[2], out[3]], axis=1)
        return jnp.pad(full, ((0, npad - n), (0, 0)))

    agg1 = run_agg(y1)
    y2 = _make_layer2(npad)(agg1, b1r, W2)
    agg2 = run_agg(y2)
    emb, z = _make_head(npad)(agg2, b2r, Wp1, bp1r, Wp2, bp2r)

    return (emb[:n], z[:n])


# speedup vs baseline: 14.6807x; 1.9470x over previous
import functools

import jax
import jax.numpy as jnp
from jax import lax
from jax.experimental import pallas as pl
from jax.experimental.pallas import tpu as pltpu
from jax.experimental.pallas import tpu_sc as plsc

NC = 2
NS = 16
NW = NC * NS
C = 80
D = 128
ROWBLK = 1024


def _sc_mesh():
    return plsc.VectorSubcoreMesh(core_axis_name="c", subcore_axis_name="s")


def _make_deg_kernel(npad, chunks):
    n_per_tile = npad // NS

    @functools.partial(
        pl.kernel,
        out_type=jax.ShapeDtypeStruct((NC * npad,), jnp.float32),
        mesh=_sc_mesh(),
        scratch_types=[
            pltpu.VMEM((chunks, C), jnp.int32),
            pltpu.VMEM((chunks, C), jnp.float32),
            pltpu.VMEM((n_per_tile,), jnp.float32),
            pltpu.VMEM_SHARED((npad,), jnp.float32),
            pltpu.SemaphoreType.DMA,
        ],
        compiler_params=pltpu.CompilerParams(
            needs_layout_passes=False, use_tc_tiling_on_sc=False),
    )
    def deg_kernel(dst_hbm, w_hbm, out_hbm, dst_v, w_v, zbuf, acc, sem):
        c = lax.axis_index("c")
        s = lax.axis_index("s")
        b = c * NS + s

        pltpu.sync_copy(dst_hbm.at[b], dst_v)
        pltpu.sync_copy(w_hbm.at[b], w_v)

        def zstore(i, _):
            zbuf[pl.ds(i * 16, 16)] = jnp.zeros((16,), jnp.float32)
            return 0
        lax.fori_loop(0, n_per_tile // 16, zstore, 0)
        pltpu.sync_copy(zbuf, acc.at[pl.ds(s * n_per_tile, n_per_tile)])
        plsc.subcore_barrier()

        def chunk(g, _):
            pltpu.async_copy(w_v.at[g], acc.at[dst_v.at[g]], sem, add=True)
            return 0
        lax.fori_loop(0, chunks, chunk, 0)

        def drain(g, _):
            pltpu.make_async_copy(w_v.at[0], acc.at[dst_v.at[0]], sem).wait()
            return 0
        lax.fori_loop(0, chunks, drain, 0)
        plsc.subcore_barrier()

        pltpu.sync_copy(acc.at[pl.ds(s * n_per_tile, n_per_tile)],
                        out_hbm.at[pl.ds(c * npad + s * n_per_tile, n_per_tile)])

    return deg_kernel


def _make_agg_kernel(npad, nacc, chunks):
    n_per_tile = nacc // NS
    dq = D // 4
    zrows = n_per_tile // 5

    @functools.partial(
        pl.kernel,
        out_type=jax.ShapeDtypeStruct((4 * nacc, dq), jnp.float32),
        mesh=_sc_mesh(),
        scratch_types=[
            pltpu.VMEM((chunks, C), jnp.int32),
            pltpu.VMEM((chunks, C), jnp.int32),
            pltpu.VMEM((chunks, C), jnp.float32),
            pltpu.VMEM((chunks, C), jnp.int32),
            pltpu.VMEM((npad,), jnp.float32),
            pltpu.VMEM((C, dq), jnp.float32),
            pltpu.VMEM((C, dq), jnp.float32),
            pltpu.VMEM((C, dq), jnp.float32),
            pltpu.VMEM((C, dq), jnp.float32),
            pltpu.VMEM((zrows, dq), jnp.float32),
            pltpu.VMEM_SHARED((nacc, dq), jnp.float32),
            pltpu.SemaphoreType.DMA,
            pltpu.SemaphoreType.DMA,
        ],
        compiler_params=pltpu.CompilerParams(
            needs_layout_passes=False, use_tc_tiling_on_sc=False),
    )
    def agg_kernel(yq_hbm, src_hbm, dst_hbm, w_hbm, dinv_hbm, out_hbm,
                   src_v, dst_v, w_v, srcq_v, dinv_v,
                   rows0, rows1, rows2, rows3, zbuf, acc, gsem, ssem):
        rows = [rows0, rows1, rows2, rows3]
        c = lax.axis_index("c")
        s = lax.axis_index("s")

        pltpu.sync_copy(src_hbm.at[s], src_v)
        pltpu.sync_copy(dst_hbm.at[s], dst_v)
        pltpu.sync_copy(w_hbm.at[s], w_v)
        pltpu.sync_copy(dinv_hbm, dinv_v)

        def zstore(i, _):
            for j in range(dq // 16):
                zbuf[i, pl.ds(j * 16, 16)] = jnp.zeros((16,), jnp.float32)
            return 0
        lax.fori_loop(0, zrows, zstore, 0)

        def gstart(g, buf):
            pltpu.async_copy(yq_hbm.at[srcq_v.at[g]], buf, gsem)

        def gwait(buf):
            pltpu.make_async_copy(yq_hbm.at[srcq_v.at[0]], buf, gsem).wait()

        def sstart(g, buf):
            pltpu.async_copy(buf, acc.at[dst_v.at[g]], ssem, add=True)

        def swait(buf):
            pltpu.make_async_copy(buf, acc.at[dst_v.at[0]], ssem).wait()

        def scale(g, buf):
            def group(k, _):
                s16 = src_v[g, pl.ds(k * 16, 16)]
                d16 = dst_v[g, pl.ds(k * 16, 16)]
                w16 = w_v[g, pl.ds(k * 16, 16)]
                n16 = (plsc.load_gather(dinv_v, [s16]) * w16 *
                       plsc.load_gather(dinv_v, [d16]))
                for i in range(16):
                    nv = n16[i]
                    row = k * 16 + i
                    for j in range(dq // 16):
                        buf[row, pl.ds(j * 16, 16)] = (
                            buf[row, pl.ds(j * 16, 16)] * nv)
                return 0
            lax.fori_loop(0, C // 16, group, 0)

        for p in range(2):
            qoff = (2 * p + c) * npad

            def shift(g, _):
                for k in range(C // 16):
                    srcq_v[g, pl.ds(k * 16, 16)] = (
                        src_v[g, pl.ds(k * 16, 16)] + qoff)
                return 0
            lax.fori_loop(0, chunks, shift, 0)

            def zcopy(k, _):
                pltpu.sync_copy(
                    zbuf, acc.at[pl.ds(s * n_per_tile + k * zrows, zrows)])
                return 0
            lax.fori_loop(0, 5, zcopy, 0)
            plsc.subcore_barrier()

            gstart(0, rows[0])
            gstart(1, rows[1])
            for g in range(2):
                gwait(rows[g])
                gstart(g + 2, rows[g + 2])
                scale(g, rows[g])
                sstart(g, rows[g])

            def steady(i, _):
                base = 2 + 4 * i
                for bb in range(4):
                    g = base + bb
                    buf = rows[(2 + bb) % 4]
                    nbuf = rows[bb % 4]
                    gwait(buf)
                    swait(nbuf)
                    gstart(g + 2, nbuf)
                    scale(g, buf)
                    sstart(g, buf)
                return 0
            lax.fori_loop(0, (chunks - 6) // 4, steady, 0)

            for e in range(4):
                g = chunks - 4 + e
                buf = rows[g % 4]
                gwait(buf)
                swait(rows[(g + 2) % 4])
                if e < 2:
                    gstart(g + 2, rows[(g + 2) % 4])
                scale(g, buf)
                sstart(g, buf)
            swait(rows[(chunks - 2) % 4])
            swait(rows[(chunks - 1) % 4])
            plsc.subcore_barrier()

            pltpu.sync_copy(
                acc.at[pl.ds(s * n_per_tile, n_per_tile)],
                out_hbm.at[pl.ds((2 * p + c) * nacc + s * n_per_tile,
                                 n_per_tile)])
            plsc.subcore_barrier()

    return agg_kernel


def _make_prep1(npad):
    grid = npad // ROWBLK

    def body(degp_ref, x_ref, w1_ref, dinv_ref, y1_ref):
        deg = degp_ref[0] + degp_ref[1]
        dinv_ref[...] = jnp.where(deg > 0, lax.rsqrt(deg), 0.0)
        y1_ref[...] = jnp.dot(x_ref[...], w1_ref[...],
                              preferred_element_type=jnp.float32)

    return pl.pallas_call(
        body,
        grid=(grid,),
        in_specs=[
            pl.BlockSpec((NC, ROWBLK // D, D), lambda i: (0, i, 0)),
            pl.BlockSpec((ROWBLK, D), lambda i: (i, 0)),
            pl.BlockSpec((D, D), lambda i: (0, 0)),
        ],
        out_specs=[
            pl.BlockSpec((ROWBLK // D, D), lambda i: (i, 0)),
            pl.BlockSpec((ROWBLK, D), lambda i: (i, 0)),
        ],
        out_shape=[
            jax.ShapeDtypeStruct((npad // D, D), jnp.float32),
            jax.ShapeDtypeStruct((npad, D), jnp.float32),
        ],
    )


def _make_layer2(npad):
    grid = npad // ROWBLK

    def body(agg_ref, b1_ref, w2_ref, y2_ref):
        h = jax.nn.relu(agg_ref[...] + b1_ref[...])
        y2_ref[...] = jnp.dot(h, w2_ref[...],
                              preferred_element_type=jnp.float32)

    return pl.pallas_call(
        body,
        grid=(grid,),
        in_specs=[
            pl.BlockSpec((ROWBLK, D), lambda i: (i, 0)),
            pl.BlockSpec((1, D), lambda i: (0, 0)),
            pl.BlockSpec((D, D), lambda i: (0, 0)),
        ],
        out_specs=pl.BlockSpec((ROWBLK, D), lambda i: (i, 0)),
        out_shape=jax.ShapeDtypeStruct((npad, D), jnp.float32),
    )


def _make_head(npad):
    grid = npad // ROWBLK

    def body(agg_ref, b2_ref, wp1_ref, bp1_ref, wp2_ref, bp2_ref,
             emb_ref, z_ref):
        emb = agg_ref[...] + b2_ref[...]
        emb_ref[...] = emb
        t = jax.nn.relu(jnp.dot(emb, wp1_ref[...],
                                preferred_element_type=jnp.float32)
                        + bp1_ref[...])
        z_ref[...] = jnp.dot(t, wp2_ref[...],
                             preferred_element_type=jnp.float32) + bp2_ref[...]

    return pl.pallas_call(
        body,
        grid=(grid,),
        in_specs=[
            pl.BlockSpec((ROWBLK, D), lambda i: (i, 0)),
            pl.BlockSpec((1, D), lambda i: (0, 0)),
            pl.BlockSpec((D, D), lambda i: (0, 0)),
            pl.BlockSpec((1, D), lambda i: (0, 0)),
            pl.BlockSpec((D, D), lambda i: (0, 0)),
            pl.BlockSpec((1, D), lambda i: (0, 0)),
        ],
        out_specs=[
            pl.BlockSpec((ROWBLK, D), lambda i: (i, 0)),
            pl.BlockSpec((ROWBLK, D), lambda i: (i, 0)),
        ],
        out_shape=[
            jax.ShapeDtypeStruct((npad, D), jnp.float32),
            jax.ShapeDtypeStruct((npad, D), jnp.float32),
        ],
    )


def kernel(x, edge_index, edge_weight, W1, b1, W2, b2, Wp1, bp1, Wp2, bp2):
    n = x.shape[0]
    e = edge_index.shape[1]

    npad = -(-n // ROWBLK) * ROWBLK
    e2 = e + n
    chunks2 = -(-e2 // (NS * C))
    while chunks2 % 4 != 2 or chunks2 < 8:
        chunks2 += 1
    e2p = chunks2 * NS * C
    chunks = e2p // (NW * C)
    assert chunks * NW * C == e2p

    loop = jnp.arange(n, dtype=jnp.int32)
    pad_i = jnp.zeros((e2p - e2,), jnp.int32)
    pad_f = jnp.zeros((e2p - e2,), jnp.float32)
    src_flat = jnp.concatenate([edge_index[0], loop, pad_i])
    dst_flat = jnp.concatenate([edge_index[1], loop, pad_i])
    w_flat = jnp.concatenate([edge_weight, jnp.ones((n,), jnp.float32), pad_f])

    dst2 = dst_flat.reshape(NW, chunks, C)
    w2 = w_flat.reshape(NW, chunks, C)
    srcA = src_flat.reshape(NS, chunks2, C)
    dstA = dst_flat.reshape(NS, chunks2, C)
    wA = w_flat.reshape(NS, chunks2, C)

    xp = jnp.pad(x, ((0, npad - n), (0, 0)))
    b1r = b1.reshape(1, D)
    b2r = b2.reshape(1, D)
    bp1r = bp1.reshape(1, D)
    bp2r = bp2.reshape(1, D)

    deg_p = _make_deg_kernel(npad, chunks)(dst2, w2).reshape(NC, npad // D, D)
    dinv2d, y1 = _make_prep1(npad)(deg_p, xp, W1)
    dinv = dinv2d.reshape(npad)

    dq = D // 4
    agg = _make_agg_kernel(npad, n, chunks2)

    def run_agg(y):
        yq = jnp.concatenate([y[:, i * dq:(i + 1) * dq] for i in range(4)],
                             axis=0)
        out = agg(yq, srcA, dstA, wA, dinv).reshape(4, n, dq)
        full = jnp.concatenate([out[0], out[1], out[2], out[3]], axis=1)
        return jnp.pad(full, ((0, npad - n), (0, 0)))

    agg1 = run_agg(y1)
    y2 = _make_layer2(npad)(agg1, b1r, W2)
    agg2 = run_agg(y2)
    emb, z = _make_head(npad)(agg2, b2r, Wp1, bp1r, Wp2, bp2r)

    return (emb[:n], z[:n])
